# Initial kernel scaffold; baseline (speedup 1.0000x reference)
import jax
import jax.numpy as jnp
from jax.experimental import pallas as pl


def kernel(x, W, b, centers, logits_scale, centers_scale):
    raise NotImplementedError("write your pallas kernel here")


# speedup vs baseline: 6.0872x; 6.0872x over previous
import functools

import jax
import jax.numpy as jnp
from jax import lax
from jax.experimental import pallas as pl
from jax.experimental.pallas import tpu as pltpu
from jax.experimental.pallas import tpu_sc as plsc

DIM = 256
CB_SIZE = 512
NUM_CB = 8
N_OUT = CB_SIZE * NUM_CB
SCALE_SPEED = 10.0

TOK_TILE = 256

SC_CORES = 2
SC_SUBCORES = 16
NW = SC_CORES * SC_SUBCORES
CHUNK_T = 8


def _encode_body(x_ref, wt_ref, b_ref, idx_ref):
    logits = jnp.dot(x_ref[...], wt_ref[...], preferred_element_type=jnp.float32)
    logits = logits + b_ref[...]
    cols = []
    for c in range(NUM_CB):
        chunk = logits[:, c * CB_SIZE:(c + 1) * CB_SIZE]
        m = jnp.max(chunk, axis=1, keepdims=True)
        ii = lax.broadcasted_iota(jnp.int32, chunk.shape, 1)
        idx = jnp.min(jnp.where(chunk == m, ii, CB_SIZE), axis=1, keepdims=True)
        cols.append(idx + c * CB_SIZE)
    idx_ref[...] = jnp.concatenate(cols, axis=1)


def _encode(x_flat, Wt, b2):
    B = x_flat.shape[0]
    grid = (B // TOK_TILE,)
    return pl.pallas_call(
        _encode_body,
        grid=grid,
        in_specs=[
            pl.BlockSpec((TOK_TILE, DIM), lambda i: (i, 0)),
            pl.BlockSpec((DIM, N_OUT), lambda i: (0, 0)),
            pl.BlockSpec((1, N_OUT), lambda i: (0, 0)),
        ],
        out_specs=pl.BlockSpec((TOK_TILE, NUM_CB), lambda i: (i, 0)),
        out_shape=jax.ShapeDtypeStruct((B, NUM_CB), jnp.int32),
        compiler_params=pltpu.CompilerParams(
            dimension_semantics=("arbitrary",),
        ),
    )(x_flat, Wt, b2)


def _make_decode(B):
    tok_per_w = B // NW
    n_chunks = tok_per_w // CHUNK_T
    mesh = plsc.VectorSubcoreMesh(core_axis_name="c", subcore_axis_name="s",
                                  num_cores=SC_CORES,
                                  num_subcores=SC_SUBCORES)

    @functools.partial(
        pl.kernel,
        mesh=mesh,
        out_type=jax.ShapeDtypeStruct((B, DIM), jnp.float32),
        scratch_types=[
            pltpu.VMEM((CHUNK_T * NUM_CB,), jnp.int32),
            pltpu.VMEM((CHUNK_T * NUM_CB, DIM), jnp.float32),
            pltpu.VMEM((CHUNK_T, DIM), jnp.float32),
            pltpu.VMEM((16,), jnp.float32),
            pltpu.SemaphoreType.DMA,
        ],
    )
    def decode(table_hbm, idx_hbm, scale_hbm, out_hbm, idx_v, rows_v, out_v,
               scale_v, sem):
        wid = lax.axis_index("s") * SC_CORES + lax.axis_index("c")
        tok0 = wid * tok_per_w
        pltpu.sync_copy(scale_hbm, scale_v)

        def chunk_body(ci, _):
            base_tok = tok0 + ci * CHUNK_T
            pltpu.sync_copy(idx_hbm.at[pl.ds(base_tok * NUM_CB,
                                             CHUNK_T * NUM_CB)], idx_v)
            pltpu.async_copy(table_hbm.at[idx_v], rows_v, sem).wait()
            scale = scale_v[...]
            for t in range(CHUNK_T):
                for j in range(DIM // 16):
                    sl = pl.ds(j * 16, 16)
                    acc = rows_v[t * NUM_CB, sl]
                    for cc in range(1, NUM_CB):
                        acc = acc + rows_v[t * NUM_CB + cc, sl]
                    out_v[t, sl] = acc * scale
            pltpu.sync_copy(out_v, out_hbm.at[pl.ds(base_tok, CHUNK_T)])
            return 0

        lax.fori_loop(0, n_chunks, chunk_body, 0)

    return decode


def kernel(x, W, b, centers, logits_scale, centers_scale):
    orig_lead = x.shape[:-1]
    x_flat = x.reshape(-1, DIM)
    B = x_flat.shape[0]
    Wt = W.T
    b2 = b.reshape(1, N_OUT)
    idx = _encode(x_flat, Wt, b2)
    idx_flat = idx.reshape(B * NUM_CB)
    table = centers.reshape(N_OUT, DIM)
    scale_vec = jnp.full((16,), jnp.exp(centers_scale * SCALE_SPEED),
                         dtype=jnp.float32)
    out = _make_decode(B)(table, idx_flat, scale_vec)
    return out.reshape(*orig_lead, DIM)
